# baseline (speedup 1.0000x reference)
import functools

import jax
import jax.numpy as jnp
from jax import lax
from jax.experimental import pallas as pl
from jax.experimental.pallas import tpu as pltpu
from jax.experimental.pallas import tpu_sc as plsc

NC = 2
NS = 16
NT = NC * NS
LANES = 16
EB = 128
D = 128
RB = 1024


def _zero_fill(ref, n):
    for i in range(n // LANES):
        ref[pl.ds(i * LANES, LANES)] = jnp.zeros((LANES,), jnp.float32)


def _mesh():
    return plsc.VectorSubcoreMesh(
        core_axis_name="c", subcore_axis_name="s", num_cores=NC, num_subcores=NS
    )


@functools.lru_cache(maxsize=None)
def _deg_kernel(np_, nblk):
    rows_per = np_ // NS

    @functools.partial(
        pl.kernel,
        out_type=jax.ShapeDtypeStruct((NC, np_), jnp.float32),
        mesh=_mesh(),
        scratch_types=[
            pltpu.VMEM((nblk, EB), jnp.int32),
            pltpu.VMEM((EB,), jnp.float32),
            pltpu.VMEM((rows_per,), jnp.float32),
            pltpu.VMEM_SHARED((np_,), jnp.float32),
        ],
    )
    def k(dst_hbm, degp_hbm, didx, ones_v, zb, acc_sh):
        c = lax.axis_index("c")
        s = lax.axis_index("s")
        g = c * NS + s

        for i in range(EB // LANES):
            ones_v[pl.ds(i * LANES, LANES)] = jnp.ones((LANES,), jnp.float32)
        _zero_fill(zb, rows_per)
        pltpu.sync_copy(zb, acc_sh.at[pl.ds(s * rows_per, rows_per)])
        pltpu.sync_copy(dst_hbm.at[g], didx)
        plsc.subcore_barrier()

        def step(j, carry):
            pltpu.sync_copy(ones_v, acc_sh.at[didx.at[j]], add=True)
            return carry

        lax.fori_loop(0, nblk, step, 0)
        plsc.subcore_barrier()

        pltpu.sync_copy(acc_sh.at[pl.ds(s * rows_per, rows_per)], zb)
        pltpu.sync_copy(zb, degp_hbm.at[c].at[pl.ds(s * rows_per, rows_per)])

    return k


@functools.lru_cache(maxsize=None)
def _agg_kernel(np_, nblk):
    rows_per = np_ // NS
    BR = 64

    @functools.partial(
        pl.kernel,
        out_type=jax.ShapeDtypeStruct((NC, np_, D), jnp.float32),
        mesh=_mesh(),
        scratch_types=[
            pltpu.VMEM((nblk, EB), jnp.int32),
            pltpu.VMEM((nblk, EB), jnp.int32),
            pltpu.VMEM((EB, D), jnp.float32),
            pltpu.VMEM((BR, D), jnp.float32),
            pltpu.VMEM_SHARED((np_, D), jnp.float32),
            pltpu.SemaphoreType.DMA,
        ],
    )
    def k(y_hbm, src_hbm, dst_hbm, aggp_hbm, sidx, didx, rows, zb, acc_sh, sem):
        c = lax.axis_index("c")
        s = lax.axis_index("s")
        g = c * NS + s

        for r in range(BR):
            for q in range(D // LANES):
                zb[r, pl.ds(q * LANES, LANES)] = jnp.zeros((LANES,), jnp.float32)
        for t in range(rows_per // BR):
            pltpu.sync_copy(zb, acc_sh.at[pl.ds(s * rows_per + t * BR, BR)])
        pltpu.sync_copy(src_hbm.at[g], sidx)
        pltpu.sync_copy(dst_hbm.at[g], didx)
        plsc.subcore_barrier()

        def step(j, carry):
            pltpu.async_copy(y_hbm.at[sidx.at[j]], rows, sem).wait()
            pltpu.sync_copy(rows, acc_sh.at[didx.at[j]], add=True)
            return carry

        lax.fori_loop(0, nblk, step, 0)
        plsc.subcore_barrier()

        base = s * rows_per
        for t in range(rows_per // BR):
            pltpu.sync_copy(acc_sh.at[pl.ds(base + t * BR, BR)], zb)
            pltpu.sync_copy(zb, aggp_hbm.at[c].at[pl.ds(base + t * BR, BR)])

    return k


@functools.lru_cache(maxsize=None)
def _sagg_kernel(np_, nblk):
    rows_per = np_ // NS

    @functools.partial(
        pl.kernel,
        out_type=jax.ShapeDtypeStruct((NC, np_), jnp.float32),
        mesh=_mesh(),
        scratch_types=[
            pltpu.VMEM((nblk, EB), jnp.int32),
            pltpu.VMEM((nblk, EB), jnp.int32),
            pltpu.VMEM((EB,), jnp.float32),
            pltpu.VMEM((rows_per,), jnp.float32),
            pltpu.VMEM_SHARED((np_,), jnp.float32),
            pltpu.SemaphoreType.DMA,
        ],
    )
    def k(zs_hbm, src_hbm, dst_hbm, outp_hbm, sidx, didx, vals, zb, acc_sh,
          gsem):
        c = lax.axis_index("c")
        s = lax.axis_index("s")
        g = c * NS + s

        pltpu.sync_copy(src_hbm.at[g], sidx)
        pltpu.sync_copy(dst_hbm.at[g], didx)
        _zero_fill(zb, rows_per)
        pltpu.sync_copy(zb, acc_sh.at[pl.ds(s * rows_per, rows_per)])
        plsc.subcore_barrier()

        def step(j, carry):
            pltpu.async_copy(zs_hbm.at[sidx.at[j]], vals, gsem).wait()
            pltpu.sync_copy(vals, acc_sh.at[didx.at[j]], add=True)
            return carry

        lax.fori_loop(0, nblk, step, 0)
        plsc.subcore_barrier()

        pltpu.sync_copy(acc_sh.at[pl.ds(s * rows_per, rows_per)], zb)
        pltpu.sync_copy(zb, outp_hbm.at[c].at[pl.ds(s * rows_per, rows_per)])

    return k


def _tc1_body(x_ref, w_ref, degp_ref, y_ref, dinv_ref):
    d = degp_ref[0] + degp_ref[1] + 1.0
    di = lax.rsqrt(d)
    dinv_ref[...] = di
    xw = jnp.dot(x_ref[...], w_ref[...], preferred_element_type=jnp.float32)
    y_ref[...] = xw * di


def _tc2_body(n, aggp_ref, y_ref, dinv_ref, b1_ref, w2_ref, zs_ref):
    a = aggp_ref[0] + aggp_ref[1] + y_ref[...]
    di = dinv_ref[...]
    h = jnp.maximum(di * a + b1_ref[...], 0.0)
    z = jnp.dot(h, w2_ref[...], preferred_element_type=jnp.float32)
    row = pl.program_id(0) * RB + lax.broadcasted_iota(jnp.int32, (RB, 1), 0)
    mask = (row < n).astype(jnp.float32)
    zs_ref[...] = di * z * mask


def _tc3_body(saggp_ref, zs_ref, dinv_ref, b2_ref, out_ref):
    ssum = saggp_ref[0] + saggp_ref[1] + zs_ref[...]
    out_ref[...] = dinv_ref[...] * ssum + b2_ref[...]


def kernel(x, edge_index, W1, b1, W2, b2):
    n, d_in = x.shape
    e = edge_index.shape[1]
    np_ = ((n + 2047) // 2048) * 2048
    epb = NT * EB
    nblka = (e + epb - 1) // epb
    nblka += nblka % 2
    epad = nblka * epb

    src = edge_index[0]
    dst = edge_index[1]
    pad = jnp.full((epad - e,), n, dtype=src.dtype)
    src_a = jnp.concatenate([src, pad]).reshape(NT, nblka, EB)
    dst_a = jnp.concatenate([dst, pad]).reshape(NT, nblka, EB)
    x_pad = jnp.zeros((np_, d_in), x.dtype).at[:n].set(x)

    degp = _deg_kernel(np_, nblka)(dst_a)

    np2 = np_ // RB
    y, dinv = pl.pallas_call(
        _tc1_body,
        grid=(np2,),
        in_specs=[
            pl.BlockSpec((RB, d_in), lambda i: (i, 0)),
            pl.BlockSpec((d_in, D), lambda i: (0, 0)),
            pl.BlockSpec((NC, RB, 1), lambda i: (0, i, 0)),
        ],
        out_specs=[
            pl.BlockSpec((RB, D), lambda i: (i, 0)),
            pl.BlockSpec((RB, 1), lambda i: (i, 0)),
        ],
        out_shape=[
            jax.ShapeDtypeStruct((np_, D), jnp.float32),
            jax.ShapeDtypeStruct((np_, 1), jnp.float32),
        ],
    )(x_pad, W1, degp.reshape(NC, np_, 1))

    aggp = _agg_kernel(np_, nblka)(y, src_a, dst_a)

    zs_col = pl.pallas_call(
        functools.partial(_tc2_body, n),
        grid=(np2,),
        in_specs=[
            pl.BlockSpec((NC, RB, D), lambda i: (0, i, 0)),
            pl.BlockSpec((RB, D), lambda i: (i, 0)),
            pl.BlockSpec((RB, 1), lambda i: (i, 0)),
            pl.BlockSpec((1, D), lambda i: (0, 0)),
            pl.BlockSpec((D, 1), lambda i: (0, 0)),
        ],
        out_specs=pl.BlockSpec((RB, 1), lambda i: (i, 0)),
        out_shape=jax.ShapeDtypeStruct((np_, 1), jnp.float32),
    )(aggp, y, dinv, b1.reshape(1, D), W2)

    saggp = _sagg_kernel(np_, nblka)(zs_col.reshape(np_), src_a, dst_a)

    out_col = pl.pallas_call(
        _tc3_body,
        in_specs=[
            pl.BlockSpec((NC, np_, 1), lambda: (0, 0, 0)),
            pl.BlockSpec((np_, 1), lambda: (0, 0)),
            pl.BlockSpec((np_, 1), lambda: (0, 0)),
            pl.BlockSpec((1, 1), lambda: (0, 0)),
        ],
        out_specs=pl.BlockSpec((np_, 1), lambda: (0, 0)),
        out_shape=jax.ShapeDtypeStruct((np_, 1), jnp.float32),
    )(saggp.reshape(NC, np_, 1), zs_col, dinv, b2.reshape(1, 1))

    return out_col[:n, 0]


# speedup vs baseline: 2.0097x; 2.0097x over previous
import functools

import jax
import jax.numpy as jnp
from jax import lax
from jax.experimental import pallas as pl
from jax.experimental.pallas import tpu as pltpu
from jax.experimental.pallas import tpu_sc as plsc

NC = 2
NS = 16
NT = NC * NS
LANES = 16
EB = 128
D = 128
RB = 1024


def _zero_fill(ref, n):
    for i in range(n // LANES):
        ref[pl.ds(i * LANES, LANES)] = jnp.zeros((LANES,), jnp.float32)


def _mesh():
    return plsc.VectorSubcoreMesh(
        core_axis_name="c", subcore_axis_name="s", num_cores=NC, num_subcores=NS
    )


@functools.lru_cache(maxsize=None)
def _deg_kernel(np_, nblk):
    rows_per = np_ // NS

    @functools.partial(
        pl.kernel,
        out_type=jax.ShapeDtypeStruct((NC, np_), jnp.float32),
        mesh=_mesh(),
        scratch_types=[
            pltpu.VMEM((nblk, EB), jnp.int32),
            pltpu.VMEM((EB,), jnp.float32),
            pltpu.VMEM((rows_per,), jnp.float32),
            pltpu.VMEM_SHARED((np_,), jnp.float32),
        ],
    )
    def k(dst_hbm, degp_hbm, didx, ones_v, zb, acc_sh):
        c = lax.axis_index("c")
        s = lax.axis_index("s")
        g = c * NS + s

        for i in range(EB // LANES):
            ones_v[pl.ds(i * LANES, LANES)] = jnp.ones((LANES,), jnp.float32)
        _zero_fill(zb, rows_per)
        pltpu.sync_copy(zb, acc_sh.at[pl.ds(s * rows_per, rows_per)])
        pltpu.sync_copy(dst_hbm.at[g], didx)
        plsc.subcore_barrier()

        def step(j, carry):
            pltpu.sync_copy(ones_v, acc_sh.at[didx.at[j]], add=True)
            return carry

        lax.fori_loop(0, nblk, step, 0)
        plsc.subcore_barrier()

        pltpu.sync_copy(acc_sh.at[pl.ds(s * rows_per, rows_per)], zb)
        pltpu.sync_copy(zb, degp_hbm.at[c].at[pl.ds(s * rows_per, rows_per)])

    return k


@functools.lru_cache(maxsize=None)
def _agg_kernel(np_, nblk):
    rows_per = np_ // NS
    BR = 64

    @functools.partial(
        pl.kernel,
        out_type=jax.ShapeDtypeStruct((NC, np_, D), jnp.float32),
        mesh=_mesh(),
        scratch_types=[
            pltpu.VMEM((nblk, EB), jnp.int32),
            pltpu.VMEM((nblk, EB), jnp.int32),
            pltpu.VMEM((EB, D), jnp.float32),
            pltpu.VMEM((BR, D), jnp.float32),
            pltpu.VMEM_SHARED((np_, D), jnp.float32),
            pltpu.SemaphoreType.DMA,
        ],
    )
    def k(y_hbm, src_hbm, dst_hbm, aggp_hbm, sidx, didx, rows, zb, acc_sh, sem):
        c = lax.axis_index("c")
        s = lax.axis_index("s")
        g = c * NS + s

        for r in range(BR):
            for q in range(D // LANES):
                zb[r, pl.ds(q * LANES, LANES)] = jnp.zeros((LANES,), jnp.float32)
        for t in range(rows_per // BR):
            pltpu.sync_copy(zb, acc_sh.at[pl.ds(s * rows_per + t * BR, BR)])
        pltpu.sync_copy(src_hbm.at[g], sidx)
        pltpu.sync_copy(dst_hbm.at[g], didx)
        plsc.subcore_barrier()

        def step(j, carry):
            pltpu.async_copy(y_hbm.at[sidx.at[j]], rows, sem).wait()
            pltpu.sync_copy(rows, acc_sh.at[didx.at[j]], add=True)
            return carry

        lax.fori_loop(0, nblk, step, 0)
        plsc.subcore_barrier()

        base = s * rows_per
        for t in range(rows_per // BR):
            pltpu.sync_copy(acc_sh.at[pl.ds(base + t * BR, BR)], zb)
            pltpu.sync_copy(zb, aggp_hbm.at[c].at[pl.ds(base + t * BR, BR)])

    return k


@functools.lru_cache(maxsize=None)
def _sagg_kernel(np_, nblk):
    rows_per = np_ // NS

    @functools.partial(
        pl.kernel,
        out_type=jax.ShapeDtypeStruct((NC, np_), jnp.float32),
        mesh=_mesh(),
        scratch_types=[
            pltpu.VMEM((nblk, EB), jnp.int32),
            pltpu.VMEM((nblk, EB), jnp.int32),
            pltpu.VMEM((EB,), jnp.float32),
            pltpu.VMEM((rows_per,), jnp.float32),
            pltpu.VMEM_SHARED((np_,), jnp.float32),
            pltpu.SemaphoreType.DMA,
        ],
    )
    def k(zs_hbm, src_hbm, dst_hbm, outp_hbm, sidx, didx, vals, zb, acc_sh,
          gsem):
        c = lax.axis_index("c")
        s = lax.axis_index("s")
        g = c * NS + s

        pltpu.sync_copy(src_hbm.at[g], sidx)
        pltpu.sync_copy(dst_hbm.at[g], didx)
        _zero_fill(zb, rows_per)
        pltpu.sync_copy(zb, acc_sh.at[pl.ds(s * rows_per, rows_per)])
        plsc.subcore_barrier()

        def step(j, carry):
            pltpu.async_copy(zs_hbm.at[sidx.at[j]], vals, gsem).wait()
            pltpu.sync_copy(vals, acc_sh.at[didx.at[j]], add=True)
            return carry

        lax.fori_loop(0, nblk, step, 0)
        plsc.subcore_barrier()

        pltpu.sync_copy(acc_sh.at[pl.ds(s * rows_per, rows_per)], zb)
        pltpu.sync_copy(zb, outp_hbm.at[c].at[pl.ds(s * rows_per, rows_per)])

    return k


def _tc1_body(x_ref, w_ref, degp_ref, y_ref, dinv_ref):
    d = degp_ref[0] + degp_ref[1] + 1.0
    di = lax.rsqrt(d)
    dinv_ref[...] = di
    xw = jnp.dot(x_ref[...], w_ref[...], preferred_element_type=jnp.float32)
    y_ref[...] = xw * di


def _tc2_body(n, aggp_ref, y_ref, dinv_ref, b1_ref, w2_ref, zs_ref):
    a = aggp_ref[0] + aggp_ref[1] + y_ref[...]
    di = dinv_ref[...]
    h = jnp.maximum(di * a + b1_ref[...], 0.0)
    z = jnp.dot(h, w2_ref[...], preferred_element_type=jnp.float32)
    row = pl.program_id(0) * RB + lax.broadcasted_iota(jnp.int32, (RB, 1), 0)
    mask = (row < n).astype(jnp.float32)
    zs_ref[...] = di * z * mask


def _tc3_body(saggp_ref, zs_ref, dinv_ref, b2_ref, out_ref):
    ssum = saggp_ref[0] + saggp_ref[1] + zs_ref[...]
    out_ref[...] = dinv_ref[...] * ssum + b2_ref[...]


def kernel(x, edge_index, W1, b1, W2, b2):
    n, d_in = x.shape
    e = edge_index.shape[1]
    np_ = ((n + 2047) // 2048) * 2048
    epb = NT * EB
    nblka = (e + epb - 1) // epb
    nblka += nblka % 2
    epad = nblka * epb

    src = edge_index[0]
    dst = edge_index[1]
    pad = n + (jnp.arange(epad - e, dtype=src.dtype) % (np_ - n))
    src_a = jnp.concatenate([src, pad]).reshape(NT, nblka, EB)
    dst_a = jnp.concatenate([dst, pad]).reshape(NT, nblka, EB)
    x_pad = jnp.zeros((np_, d_in), x.dtype).at[:n].set(x)

    degp = _deg_kernel(np_, nblka)(dst_a)

    np2 = np_ // RB
    y, dinv = pl.pallas_call(
        _tc1_body,
        grid=(np2,),
        in_specs=[
            pl.BlockSpec((RB, d_in), lambda i: (i, 0)),
            pl.BlockSpec((d_in, D), lambda i: (0, 0)),
            pl.BlockSpec((NC, RB, 1), lambda i: (0, i, 0)),
        ],
        out_specs=[
            pl.BlockSpec((RB, D), lambda i: (i, 0)),
            pl.BlockSpec((RB, 1), lambda i: (i, 0)),
        ],
        out_shape=[
            jax.ShapeDtypeStruct((np_, D), jnp.float32),
            jax.ShapeDtypeStruct((np_, 1), jnp.float32),
        ],
    )(x_pad, W1, degp.reshape(NC, np_, 1))

    aggp = _agg_kernel(np_, nblka)(y, src_a, dst_a)

    zs_col = pl.pallas_call(
        functools.partial(_tc2_body, n),
        grid=(np2,),
        in_specs=[
            pl.BlockSpec((NC, RB, D), lambda i: (0, i, 0)),
            pl.BlockSpec((RB, D), lambda i: (i, 0)),
            pl.BlockSpec((RB, 1), lambda i: (i, 0)),
            pl.BlockSpec((1, D), lambda i: (0, 0)),
            pl.BlockSpec((D, 1), lambda i: (0, 0)),
        ],
        out_specs=pl.BlockSpec((RB, 1), lambda i: (i, 0)),
        out_shape=jax.ShapeDtypeStruct((np_, 1), jnp.float32),
    )(aggp, y, dinv, b1.reshape(1, D), W2)

    saggp = _sagg_kernel(np_, nblka)(zs_col.reshape(np_), src_a, dst_a)

    out_col = pl.pallas_call(
        _tc3_body,
        in_specs=[
            pl.BlockSpec((NC, np_, 1), lambda: (0, 0, 0)),
            pl.BlockSpec((np_, 1), lambda: (0, 0)),
            pl.BlockSpec((np_, 1), lambda: (0, 0)),
            pl.BlockSpec((1, 1), lambda: (0, 0)),
        ],
        out_specs=pl.BlockSpec((np_, 1), lambda: (0, 0)),
        out_shape=jax.ShapeDtypeStruct((np_, 1), jnp.float32),
    )(saggp.reshape(NC, np_, 1), zs_col, dinv, b2.reshape(1, 1))

    return out_col[:n, 0]


# speedup vs baseline: 2.4489x; 1.2185x over previous
import functools

import jax
import jax.numpy as jnp
from jax import lax
from jax.experimental import pallas as pl
from jax.experimental.pallas import tpu as pltpu
from jax.experimental.pallas import tpu_sc as plsc

NC = 2
NS = 16
NT = NC * NS
LANES = 16
EB = 128
D = 128
RB = 1024


def _zero_fill(ref, n):
    for i in range(n // LANES):
        ref[pl.ds(i * LANES, LANES)] = jnp.zeros((LANES,), jnp.float32)


def _mesh():
    return plsc.VectorSubcoreMesh(
        core_axis_name="c", subcore_axis_name="s", num_cores=NC, num_subcores=NS
    )


@functools.lru_cache(maxsize=None)
def _deg_kernel(np_, nblk):
    rows_per = np_ // NS

    @functools.partial(
        pl.kernel,
        out_type=jax.ShapeDtypeStruct((NC, np_), jnp.float32),
        mesh=_mesh(),
        scratch_types=[
            pltpu.VMEM((nblk, EB), jnp.int32),
            pltpu.VMEM((EB,), jnp.float32),
            pltpu.VMEM((rows_per,), jnp.float32),
            pltpu.VMEM_SHARED((np_,), jnp.float32),
        ],
    )
    def k(dst_hbm, degp_hbm, didx, ones_v, zb, acc_sh):
        c = lax.axis_index("c")
        s = lax.axis_index("s")
        g = c * NS + s

        for i in range(EB // LANES):
            ones_v[pl.ds(i * LANES, LANES)] = jnp.ones((LANES,), jnp.float32)
        _zero_fill(zb, rows_per)
        pltpu.sync_copy(zb, acc_sh.at[pl.ds(s * rows_per, rows_per)])
        pltpu.sync_copy(dst_hbm.at[g], didx)
        plsc.subcore_barrier()

        def step(j, carry):
            pltpu.sync_copy(ones_v, acc_sh.at[didx.at[j]], add=True)
            return carry

        lax.fori_loop(0, nblk, step, 0)
        plsc.subcore_barrier()

        pltpu.sync_copy(acc_sh.at[pl.ds(s * rows_per, rows_per)], zb)
        pltpu.sync_copy(zb, degp_hbm.at[c].at[pl.ds(s * rows_per, rows_per)])

    return k


@functools.lru_cache(maxsize=None)
def _agg_kernel(np_, nblk):
    rows_per = np_ // NS
    BR = 16

    @functools.partial(
        pl.kernel,
        out_type=jax.ShapeDtypeStruct((NC, np_, D), jnp.float32),
        mesh=_mesh(),
        scratch_types=[
            pltpu.VMEM((nblk, EB), jnp.int32),
            pltpu.VMEM((2, EB), jnp.int32),
            pltpu.VMEM((2, EB), jnp.int32),
            pltpu.VMEM((EB, D), jnp.float32),
            pltpu.VMEM((EB, D), jnp.float32),
            pltpu.VMEM((BR, D), jnp.float32),
            pltpu.VMEM_SHARED((np_, D), jnp.float32),
            pltpu.SemaphoreType.DMA,
            pltpu.SemaphoreType.DMA,
        ],
    )
    def k(y_hbm, ei_hbm, aggp_hbm, eidx, srng, drng, buf0, buf1, zb,
          acc_sh, g0, g1):
        c = lax.axis_index("c")
        s = lax.axis_index("s")
        g = c * NS + s

        for r in range(BR):
            for q in range(D // LANES):
                zb[r, pl.ds(q * LANES, LANES)] = jnp.zeros((LANES,), jnp.float32)
        for t in range(rows_per // BR):
            pltpu.sync_copy(zb, acc_sh.at[pl.ds(s * rows_per + t * BR, BR)])
        pltpu.sync_copy(ei_hbm.at[g], eidx)
        plsc.subcore_barrier()

        bufs = (buf0, buf1)
        gsems = (g0, g1)
        mask16 = jnp.int32(0xFFFF)

        def unpack(j, p):
            for q in range(EB // LANES):
                sl = pl.ds(q * LANES, LANES)
                pk = eidx[j, sl]
                srng[p, sl] = lax.bitwise_and(pk, mask16)
                drng[p, sl] = lax.shift_right_logical(pk, 16)

        def issue(j, p):
            unpack(j, p)
            pltpu.async_copy(y_hbm.at[srng.at[p]], bufs[p], gsems[p])

        issue(0, 0)
        issue(1, 1)

        def pair(jj, carry):
            j = 2 * jj
            for p in range(2):
                pltpu.make_async_copy(y_hbm.at[pl.ds(0, EB)], bufs[p],
                                      gsems[p]).wait()
                pltpu.sync_copy(bufs[p], acc_sh.at[drng.at[p]], add=True)

                @pl.when(j + p + 2 < nblk)
                def _():
                    issue(j + p + 2, p)
            return carry

        lax.fori_loop(0, nblk // 2, pair, 0)
        plsc.subcore_barrier()

        base = s * rows_per
        for t in range(rows_per // BR):
            pltpu.sync_copy(acc_sh.at[pl.ds(base + t * BR, BR)], zb)
            pltpu.sync_copy(zb, aggp_hbm.at[c].at[pl.ds(base + t * BR, BR)])

    return k


@functools.lru_cache(maxsize=None)
def _sagg_kernel(np_, nblk):
    rows_per = np_ // NS

    @functools.partial(
        pl.kernel,
        out_type=jax.ShapeDtypeStruct((NC, np_), jnp.float32),
        mesh=_mesh(),
        scratch_types=[
            pltpu.VMEM((nblk, EB), jnp.int32),
            pltpu.VMEM((nblk, EB), jnp.int32),
            pltpu.VMEM((EB,), jnp.float32),
            pltpu.VMEM((rows_per,), jnp.float32),
            pltpu.VMEM_SHARED((np_,), jnp.float32),
            pltpu.SemaphoreType.DMA,
        ],
    )
    def k(zs_hbm, src_hbm, dst_hbm, outp_hbm, sidx, didx, vals, zb, acc_sh,
          gsem):
        c = lax.axis_index("c")
        s = lax.axis_index("s")
        g = c * NS + s

        pltpu.sync_copy(src_hbm.at[g], sidx)
        pltpu.sync_copy(dst_hbm.at[g], didx)
        _zero_fill(zb, rows_per)
        pltpu.sync_copy(zb, acc_sh.at[pl.ds(s * rows_per, rows_per)])
        plsc.subcore_barrier()

        def step(j, carry):
            pltpu.async_copy(zs_hbm.at[sidx.at[j]], vals, gsem).wait()
            pltpu.sync_copy(vals, acc_sh.at[didx.at[j]], add=True)
            return carry

        lax.fori_loop(0, nblk, step, 0)
        plsc.subcore_barrier()

        pltpu.sync_copy(acc_sh.at[pl.ds(s * rows_per, rows_per)], zb)
        pltpu.sync_copy(zb, outp_hbm.at[c].at[pl.ds(s * rows_per, rows_per)])

    return k


def _tc1_body(x_ref, w_ref, degp_ref, y_ref, dinv_ref):
    d = degp_ref[0] + degp_ref[1] + 1.0
    di = lax.rsqrt(d)
    dinv_ref[...] = di
    xw = jnp.dot(x_ref[...], w_ref[...], preferred_element_type=jnp.float32)
    y_ref[...] = xw * di


def _tc2_body(n, aggp_ref, y_ref, dinv_ref, b1_ref, w2_ref, zs_ref):
    a = aggp_ref[0] + aggp_ref[1] + y_ref[...]
    di = dinv_ref[...]
    h = jnp.maximum(di * a + b1_ref[...], 0.0)
    z = jnp.dot(h, w2_ref[...], preferred_element_type=jnp.float32)
    row = pl.program_id(0) * RB + lax.broadcasted_iota(jnp.int32, (RB, 1), 0)
    mask = (row < n).astype(jnp.float32)
    zs_ref[...] = di * z * mask


def _tc3_body(saggp_ref, zs_ref, dinv_ref, b2_ref, out_ref):
    ssum = saggp_ref[0] + saggp_ref[1] + zs_ref[...]
    out_ref[...] = dinv_ref[...] * ssum + b2_ref[...]


def kernel(x, edge_index, W1, b1, W2, b2):
    n, d_in = x.shape
    e = edge_index.shape[1]
    np_ = ((n + 2047) // 2048) * 2048
    epb = NT * EB
    nblka = (e + epb - 1) // epb
    nblka += nblka % 2
    epad = nblka * epb

    src = edge_index[0]
    dst = edge_index[1]
    pad = n + (jnp.arange(epad - e, dtype=src.dtype) % (np_ - n))
    src_p = jnp.concatenate([src, pad])
    dst_p = jnp.concatenate([dst, pad])
    src_a = src_p.reshape(NT, nblka, EB)
    dst_a = dst_p.reshape(NT, nblka, EB)
    ei_pk = (src_p | (dst_p << 16)).reshape(NT, nblka, EB)
    x_pad = jnp.zeros((np_, d_in), x.dtype).at[:n].set(x)

    degp = _deg_kernel(np_, nblka)(dst_a)

    np2 = np_ // RB
    y, dinv = pl.pallas_call(
        _tc1_body,
        grid=(np2,),
        in_specs=[
            pl.BlockSpec((RB, d_in), lambda i: (i, 0)),
            pl.BlockSpec((d_in, D), lambda i: (0, 0)),
            pl.BlockSpec((NC, RB, 1), lambda i: (0, i, 0)),
        ],
        out_specs=[
            pl.BlockSpec((RB, D), lambda i: (i, 0)),
            pl.BlockSpec((RB, 1), lambda i: (i, 0)),
        ],
        out_shape=[
            jax.ShapeDtypeStruct((np_, D), jnp.float32),
            jax.ShapeDtypeStruct((np_, 1), jnp.float32),
        ],
    )(x_pad, W1, degp.reshape(NC, np_, 1))

    aggp = _agg_kernel(np_, nblka)(y, ei_pk)

    zs_col = pl.pallas_call(
        functools.partial(_tc2_body, n),
        grid=(np2,),
        in_specs=[
            pl.BlockSpec((NC, RB, D), lambda i: (0, i, 0)),
            pl.BlockSpec((RB, D), lambda i: (i, 0)),
            pl.BlockSpec((RB, 1), lambda i: (i, 0)),
            pl.BlockSpec((1, D), lambda i: (0, 0)),
            pl.BlockSpec((D, 1), lambda i: (0, 0)),
        ],
        out_specs=pl.BlockSpec((RB, 1), lambda i: (i, 0)),
        out_shape=jax.ShapeDtypeStruct((np_, 1), jnp.float32),
    )(aggp, y, dinv, b1.reshape(1, D), W2)

    saggp = _sagg_kernel(np_, nblka)(zs_col.reshape(np_), src_a, dst_a)

    out_col = pl.pallas_call(
        _tc3_body,
        in_specs=[
            pl.BlockSpec((NC, np_, 1), lambda: (0, 0, 0)),
            pl.BlockSpec((np_, 1), lambda: (0, 0)),
            pl.BlockSpec((np_, 1), lambda: (0, 0)),
            pl.BlockSpec((1, 1), lambda: (0, 0)),
        ],
        out_specs=pl.BlockSpec((np_, 1), lambda: (0, 0)),
        out_shape=jax.ShapeDtypeStruct((np_, 1), jnp.float32),
    )(saggp.reshape(NC, np_, 1), zs_col, dinv, b2.reshape(1, 1))

    return out_col[:n, 0]


# speedup vs baseline: 2.6629x; 1.0874x over previous
import functools

import jax
import jax.numpy as jnp
from jax import lax
from jax.experimental import pallas as pl
from jax.experimental.pallas import tpu as pltpu
from jax.experimental.pallas import tpu_sc as plsc

NC = 2
NS = 16
NT = NC * NS
LANES = 16
EB = 128
D = 128
RB = 1024


def _zero_fill(ref, n):
    for i in range(n // LANES):
        ref[pl.ds(i * LANES, LANES)] = jnp.zeros((LANES,), jnp.float32)


def _mesh():
    return plsc.VectorSubcoreMesh(
        core_axis_name="c", subcore_axis_name="s", num_cores=NC, num_subcores=NS
    )


@functools.lru_cache(maxsize=None)
def _deg_kernel(np_, nblk):
    rows_per = np_ // NS

    @functools.partial(
        pl.kernel,
        out_type=jax.ShapeDtypeStruct((NC, np_), jnp.float32),
        mesh=_mesh(),
        scratch_types=[
            pltpu.VMEM((nblk, EB), jnp.int32),
            pltpu.VMEM((EB,), jnp.float32),
            pltpu.VMEM((rows_per,), jnp.float32),
            pltpu.VMEM_SHARED((np_,), jnp.float32),
            pltpu.SemaphoreType.DMA,
        ],
    )
    def k(dst_hbm, degp_hbm, didx, ones_v, zb, acc_sh, sem):
        c = lax.axis_index("c")
        s = lax.axis_index("s")
        g = c * NS + s

        for i in range(EB // LANES):
            ones_v[pl.ds(i * LANES, LANES)] = jnp.ones((LANES,), jnp.float32)
        _zero_fill(zb, rows_per)
        pltpu.sync_copy(zb, acc_sh.at[pl.ds(s * rows_per, rows_per)])
        pltpu.sync_copy(dst_hbm.at[g], didx)
        plsc.subcore_barrier()

        def fire(j, carry):
            pltpu.async_copy(ones_v, acc_sh.at[didx.at[j]], sem, add=True)
            return carry

        lax.fori_loop(0, nblk, fire, 0)

        def drain(j, carry):
            pltpu.make_async_copy(ones_v, acc_sh.at[didx.at[0]], sem).wait()
            return carry

        lax.fori_loop(0, nblk, drain, 0)
        plsc.subcore_barrier()

        pltpu.sync_copy(acc_sh.at[pl.ds(s * rows_per, rows_per)], zb)
        pltpu.sync_copy(zb, degp_hbm.at[c].at[pl.ds(s * rows_per, rows_per)])

    return k


@functools.lru_cache(maxsize=None)
def _agg_kernel(np_, nblk):
    rows_per = np_ // NS
    BR = 16

    @functools.partial(
        pl.kernel,
        out_type=jax.ShapeDtypeStruct((NC, np_, D), jnp.float32),
        mesh=_mesh(),
        scratch_types=[
            pltpu.VMEM((nblk, EB), jnp.int32),
            pltpu.VMEM((2, EB), jnp.int32),
            pltpu.VMEM((2, EB), jnp.int32),
            pltpu.VMEM((EB, D), jnp.float32),
            pltpu.VMEM((EB, D), jnp.float32),
            pltpu.VMEM((BR, D), jnp.float32),
            pltpu.VMEM_SHARED((np_, D), jnp.float32),
            pltpu.SemaphoreType.DMA,
            pltpu.SemaphoreType.DMA,
        ],
    )
    def k(y_hbm, ei_hbm, aggp_hbm, eidx, srng, drng, buf0, buf1, zb,
          acc_sh, g0, g1):
        c = lax.axis_index("c")
        s = lax.axis_index("s")
        g = c * NS + s

        for r in range(BR):
            for q in range(D // LANES):
                zb[r, pl.ds(q * LANES, LANES)] = jnp.zeros((LANES,), jnp.float32)
        for t in range(rows_per // BR):
            pltpu.sync_copy(zb, acc_sh.at[pl.ds(s * rows_per + t * BR, BR)])
        pltpu.sync_copy(ei_hbm.at[g], eidx)
        plsc.subcore_barrier()

        bufs = (buf0, buf1)
        gsems = (g0, g1)
        mask16 = jnp.int32(0xFFFF)

        def unpack(j, p):
            for q in range(EB // LANES):
                sl = pl.ds(q * LANES, LANES)
                pk = eidx[j, sl]
                srng[p, sl] = lax.bitwise_and(pk, mask16)
                drng[p, sl] = lax.shift_right_logical(pk, 16)

        def issue(j, p):
            unpack(j, p)
            pltpu.async_copy(y_hbm.at[srng.at[p]], bufs[p], gsems[p])

        issue(0, 0)
        issue(1, 1)

        def pair(jj, carry):
            j = 2 * jj
            for p in range(2):
                pltpu.make_async_copy(y_hbm.at[pl.ds(0, EB)], bufs[p],
                                      gsems[p]).wait()
                pltpu.sync_copy(bufs[p], acc_sh.at[drng.at[p]], add=True)

                @pl.when(j + p + 2 < nblk)
                def _():
                    issue(j + p + 2, p)
            return carry

        lax.fori_loop(0, nblk // 2, pair, 0)
        plsc.subcore_barrier()

        base = s * rows_per
        for t in range(rows_per // BR):
            pltpu.sync_copy(acc_sh.at[pl.ds(base + t * BR, BR)], zb)
            pltpu.sync_copy(zb, aggp_hbm.at[c].at[pl.ds(base + t * BR, BR)])

    return k


@functools.lru_cache(maxsize=None)
def _sagg_kernel(np_, nblk):
    rows_per = np_ // NS

    @functools.partial(
        pl.kernel,
        out_type=jax.ShapeDtypeStruct((np_,), jnp.float32),
        mesh=_mesh(),
        scratch_types=[
            pltpu.VMEM((nblk, EB), jnp.int32),
            pltpu.VMEM((nblk, EB), jnp.int32),
            pltpu.VMEM((2, EB), jnp.float32),
            pltpu.VMEM((rows_per,), jnp.float32),
            pltpu.VMEM((rows_per,), jnp.float32),
            pltpu.VMEM((rows_per,), jnp.float32),
            pltpu.VMEM((LANES,), jnp.float32),
            pltpu.VMEM_SHARED((np_,), jnp.float32),
            pltpu.SemaphoreType.DMA,
            pltpu.SemaphoreType.DMA,
        ],
    )
    def k(zs_hbm, src_hbm, dst_hbm, dinv_hbm, b2_hbm, out_hbm, sidx, didx,
          vals, zb, dv, zv, b2v, acc_sh, g0, g1):
        c = lax.axis_index("c")
        s = lax.axis_index("s")

        @pl.when(c == 0)
        def _body():
            pltpu.sync_copy(src_hbm.at[s], sidx)
            pltpu.sync_copy(dst_hbm.at[s], didx)
            _zero_fill(zb, rows_per)
            pltpu.sync_copy(zb, acc_sh.at[pl.ds(s * rows_per, rows_per)])
            plsc.subcore_barrier()

            gsems = (g0, g1)

            def issue(j, p):
                pltpu.async_copy(zs_hbm.at[sidx.at[j]], vals.at[p], gsems[p])

            issue(0, 0)
            issue(1, 1)

            def pair(jj, carry):
                j = 2 * jj
                for p in range(2):
                    pltpu.make_async_copy(zs_hbm.at[pl.ds(0, EB)], vals.at[p],
                                          gsems[p]).wait()
                    pltpu.sync_copy(vals.at[p], acc_sh.at[didx.at[j + p]],
                                    add=True)

                    @pl.when(j + p + 2 < nblk)
                    def _():
                        issue(j + p + 2, p)
                return carry

            lax.fori_loop(0, nblk // 2, pair, 0)
            plsc.subcore_barrier()

            base = s * rows_per
            pltpu.sync_copy(acc_sh.at[pl.ds(base, rows_per)], zb)
            pltpu.sync_copy(dinv_hbm.at[pl.ds(base, rows_per)], dv)
            pltpu.sync_copy(zs_hbm.at[pl.ds(base, rows_per)], zv)
            pltpu.sync_copy(b2_hbm, b2v)
            for i in range(rows_per // LANES):
                sl = pl.ds(i * LANES, LANES)
                zv[sl] = dv[sl] * (zb[sl] + zv[sl]) + b2v[...]
            pltpu.sync_copy(zv, out_hbm.at[pl.ds(base, rows_per)])

    return k


def _tc1_body(x_ref, w_ref, degp_ref, y_ref, dinv_ref):
    d = degp_ref[0] + degp_ref[1] + 1.0
    di = lax.rsqrt(d)
    dinv_ref[...] = di
    xw = jnp.dot(x_ref[...], w_ref[...], preferred_element_type=jnp.float32)
    y_ref[...] = xw * di


def _tc2_body(n, aggp_ref, y_ref, dinv_ref, b1_ref, w2_ref, zs_ref):
    a = aggp_ref[0] + aggp_ref[1] + y_ref[...]
    di = dinv_ref[...]
    h = jnp.maximum(di * a + b1_ref[...], 0.0)
    z = jnp.dot(h, w2_ref[...], preferred_element_type=jnp.float32)
    row = pl.program_id(0) * RB + lax.broadcasted_iota(jnp.int32, (RB, 1), 0)
    mask = (row < n).astype(jnp.float32)
    zs_ref[...] = di * z * mask


def kernel(x, edge_index, W1, b1, W2, b2):
    n, d_in = x.shape
    e = edge_index.shape[1]
    np_ = ((n + 2047) // 2048) * 2048
    epb = NT * EB
    nblka = (e + epb - 1) // epb
    nblka += nblka % 2
    epad = nblka * epb

    src = edge_index[0]
    dst = edge_index[1]
    pad = n + (jnp.arange(epad - e, dtype=src.dtype) % (np_ - n))
    src_p = jnp.concatenate([src, pad])
    dst_p = jnp.concatenate([dst, pad])
    src_a = src_p.reshape(NT, nblka, EB)
    dst_a = dst_p.reshape(NT, nblka, EB)
    ei_pk = (src_p | (dst_p << 16)).reshape(NT, nblka, EB)
    x_pad = jnp.zeros((np_, d_in), x.dtype).at[:n].set(x)

    degp = _deg_kernel(np_, nblka)(dst_a)

    np2 = np_ // RB
    y, dinv = pl.pallas_call(
        _tc1_body,
        grid=(np2,),
        in_specs=[
            pl.BlockSpec((RB, d_in), lambda i: (i, 0)),
            pl.BlockSpec((d_in, D), lambda i: (0, 0)),
            pl.BlockSpec((NC, RB, 1), lambda i: (0, i, 0)),
        ],
        out_specs=[
            pl.BlockSpec((RB, D), lambda i: (i, 0)),
            pl.BlockSpec((RB, 1), lambda i: (i, 0)),
        ],
        out_shape=[
            jax.ShapeDtypeStruct((np_, D), jnp.float32),
            jax.ShapeDtypeStruct((np_, 1), jnp.float32),
        ],
    )(x_pad, W1, degp.reshape(NC, np_, 1))

    aggp = _agg_kernel(np_, nblka)(y, ei_pk)

    zs_col = pl.pallas_call(
        functools.partial(_tc2_body, n),
        grid=(np2,),
        in_specs=[
            pl.BlockSpec((NC, RB, D), lambda i: (0, i, 0)),
            pl.BlockSpec((RB, D), lambda i: (i, 0)),
            pl.BlockSpec((RB, 1), lambda i: (i, 0)),
            pl.BlockSpec((1, D), lambda i: (0, 0)),
            pl.BlockSpec((D, 1), lambda i: (0, 0)),
        ],
        out_specs=pl.BlockSpec((RB, 1), lambda i: (i, 0)),
        out_shape=jax.ShapeDtypeStruct((np_, 1), jnp.float32),
    )(aggp, y, dinv, b1.reshape(1, D), W2)

    nblkc = 2 * nblka
    src_c = src_p.reshape(NS, nblkc, EB)
    dst_c = dst_p.reshape(NS, nblkc, EB)
    out = _sagg_kernel(np_, nblkc)(
        zs_col.reshape(np_), src_c, dst_c, dinv.reshape(np_),
        jnp.broadcast_to(b2, (LANES,)),
    )
    return out[:n]


# speedup vs baseline: 3.0534x; 1.1467x over previous
import functools

import jax
import jax.numpy as jnp
from jax import lax
from jax.experimental import pallas as pl
from jax.experimental.pallas import tpu as pltpu
from jax.experimental.pallas import tpu_sc as plsc

NC = 2
NS = 16
NT = NC * NS
LANES = 16
EB = 128
D = 128
RB = 1024


def _zero_fill(ref, n):
    for i in range(n // LANES):
        ref[pl.ds(i * LANES, LANES)] = jnp.zeros((LANES,), jnp.float32)


def _mesh():
    return plsc.VectorSubcoreMesh(
        core_axis_name="c", subcore_axis_name="s", num_cores=NC, num_subcores=NS
    )


@functools.lru_cache(maxsize=None)
def _deg_kernel(np_, nblk):
    rows_per = np_ // NS

    @functools.partial(
        pl.kernel,
        out_type=jax.ShapeDtypeStruct((NC, np_), jnp.float32),
        mesh=_mesh(),
        scratch_types=[
            pltpu.VMEM((nblk, EB), jnp.int32),
            pltpu.VMEM((EB,), jnp.float32),
            pltpu.VMEM((rows_per,), jnp.float32),
            pltpu.VMEM_SHARED((np_,), jnp.float32),
            pltpu.SemaphoreType.DMA,
        ],
    )
    def k(dst_hbm, degp_hbm, didx, ones_v, zb, acc_sh, sem):
        c = lax.axis_index("c")
        s = lax.axis_index("s")
        g = c * NS + s

        for i in range(EB // LANES):
            ones_v[pl.ds(i * LANES, LANES)] = jnp.ones((LANES,), jnp.float32)
        _zero_fill(zb, rows_per)
        pltpu.sync_copy(zb, acc_sh.at[pl.ds(s * rows_per, rows_per)])
        pltpu.sync_copy(dst_hbm.at[g], didx)
        plsc.subcore_barrier()

        def fire(j, carry):
            pltpu.async_copy(ones_v, acc_sh.at[didx.at[j]], sem, add=True)
            return carry

        lax.fori_loop(0, nblk, fire, 0)

        def drain(j, carry):
            pltpu.make_async_copy(ones_v, acc_sh.at[didx.at[0]], sem).wait()
            return carry

        lax.fori_loop(0, nblk, drain, 0)
        plsc.subcore_barrier()

        pltpu.sync_copy(acc_sh.at[pl.ds(s * rows_per, rows_per)], zb)
        pltpu.sync_copy(zb, degp_hbm.at[c].at[pl.ds(s * rows_per, rows_per)])

    return k


@functools.lru_cache(maxsize=None)
def _agg_kernel(np_, nblk):
    rows_per = np_ // NS
    BR = 16

    @functools.partial(
        pl.kernel,
        out_type=jax.ShapeDtypeStruct((NC, np_, D), jnp.float32),
        mesh=_mesh(),
        scratch_types=[
            pltpu.VMEM((nblk, EB), jnp.int32),
            pltpu.VMEM((2, EB), jnp.int32),
            pltpu.VMEM((2, EB), jnp.int32),
            pltpu.VMEM((EB, D), jnp.float32),
            pltpu.VMEM((EB, D), jnp.float32),
            pltpu.VMEM((BR, D), jnp.float32),
            pltpu.VMEM_SHARED((np_, D), jnp.float32),
            pltpu.SemaphoreType.DMA,
            pltpu.SemaphoreType.DMA,
        ],
    )
    def k(y_hbm, ei_hbm, aggp_hbm, eidx, srng, drng, buf0, buf1, zb,
          acc_sh, g0, g1):
        c = lax.axis_index("c")
        s = lax.axis_index("s")
        g = c * NS + s

        for r in range(BR):
            for q in range(D // LANES):
                zb[r, pl.ds(q * LANES, LANES)] = jnp.zeros((LANES,), jnp.float32)
        for t in range(rows_per // BR):
            pltpu.sync_copy(zb, acc_sh.at[pl.ds(s * rows_per + t * BR, BR)])
        pltpu.sync_copy(ei_hbm.at[g], eidx)
        plsc.subcore_barrier()

        bufs = (buf0, buf1)
        gsems = (g0, g1)
        mask16 = jnp.int32(0xFFFF)

        def unpack(j, p):
            for q in range(EB // LANES):
                sl = pl.ds(q * LANES, LANES)
                pk = eidx[j, sl]
                srng[p, sl] = lax.bitwise_and(pk, mask16)
                drng[p, sl] = lax.shift_right_logical(pk, 16)

        def issue(j, p):
            unpack(j, p)
            pltpu.async_copy(y_hbm.at[srng.at[p]], bufs[p], gsems[p])

        issue(0, 0)
        issue(1, 1)

        def pair(jj, carry):
            j = 2 * jj
            for p in range(2):
                pltpu.make_async_copy(y_hbm.at[pl.ds(0, EB)], bufs[p],
                                      gsems[p]).wait()
                pltpu.sync_copy(bufs[p], acc_sh.at[drng.at[p]], add=True)

                @pl.when(j + p + 2 < nblk)
                def _():
                    issue(j + p + 2, p)
            return carry

        lax.fori_loop(0, nblk // 2, pair, 0)
        plsc.subcore_barrier()

        base = s * rows_per
        for t in range(rows_per // BR):
            pltpu.sync_copy(acc_sh.at[pl.ds(base + t * BR, BR)], zb)
            pltpu.sync_copy(zb, aggp_hbm.at[c].at[pl.ds(base + t * BR, BR)])

    return k


@functools.lru_cache(maxsize=None)
def _sagg_kernel(np_, nblk):
    rows_per = np_ // NS

    @functools.partial(
        pl.kernel,
        out_type=jax.ShapeDtypeStruct((np_,), jnp.float32),
        mesh=_mesh(),
        scratch_types=[
            pltpu.VMEM((nblk, EB), jnp.int32),
            pltpu.VMEM((nblk, EB), jnp.int32),
            pltpu.VMEM((8, EB), jnp.float32),
            pltpu.VMEM((rows_per,), jnp.float32),
            pltpu.VMEM((rows_per,), jnp.float32),
            pltpu.VMEM((rows_per,), jnp.float32),
            pltpu.VMEM((LANES,), jnp.float32),
            pltpu.VMEM_SHARED((np_,), jnp.float32),
        ] + [pltpu.SemaphoreType.DMA] * 16,
    )
    def k(zs_hbm, src_hbm, dst_hbm, dinv_hbm, b2_hbm, out_hbm, sidx, didx,
          vals, zb, dv, zv, b2v, acc_sh, *sems):
        c = lax.axis_index("c")
        s = lax.axis_index("s")
        gsems = sems[:8]
        ssems = sems[8:]

        @pl.when(c == 0)
        def _body():
            pltpu.sync_copy(src_hbm.at[s], sidx)
            pltpu.sync_copy(dst_hbm.at[s], didx)
            _zero_fill(zb, rows_per)
            pltpu.sync_copy(zb, acc_sh.at[pl.ds(s * rows_per, rows_per)])
            plsc.subcore_barrier()

            def issue(j, p):
                pltpu.async_copy(zs_hbm.at[sidx.at[j]], vals.at[p], gsems[p])

            for p in range(4):
                issue(p, p)

            def octet(jj, carry):
                j0 = 8 * jj
                for p in range(8):
                    j = j0 + p
                    pltpu.make_async_copy(zs_hbm.at[pl.ds(0, EB)], vals.at[p],
                                          gsems[p]).wait()
                    pltpu.async_copy(vals.at[p], acc_sh.at[didx.at[j]],
                                     ssems[p], add=True)
                    q = (p + 4) % 8

                    @pl.when(j + 4 < nblk)
                    def _():
                        @pl.when(j >= 4)
                        def _():
                            pltpu.make_async_copy(
                                zs_hbm.at[pl.ds(0, EB)], vals.at[q],
                                ssems[q]).wait()

                        issue(j + 4, q)
                return carry

            lax.fori_loop(0, nblk // 8, octet, 0)
            for p in range(8):
                pltpu.make_async_copy(zs_hbm.at[pl.ds(0, EB)], vals.at[p],
                                      ssems[p]).wait()
            plsc.subcore_barrier()

            base = s * rows_per
            pltpu.sync_copy(acc_sh.at[pl.ds(base, rows_per)], zb)
            pltpu.sync_copy(dinv_hbm.at[pl.ds(base, rows_per)], dv)
            pltpu.sync_copy(zs_hbm.at[pl.ds(base, rows_per)], zv)
            pltpu.sync_copy(b2_hbm, b2v)
            for i in range(rows_per // LANES):
                sl = pl.ds(i * LANES, LANES)
                zv[sl] = dv[sl] * (zb[sl] + zv[sl]) + b2v[...]
            pltpu.sync_copy(zv, out_hbm.at[pl.ds(base, rows_per)])

    return k


def _tc1_body(x_ref, w_ref, degp_ref, y_ref, dinv_ref):
    d = degp_ref[0] + degp_ref[1] + 1.0
    di = lax.rsqrt(d)
    dinv_ref[...] = di
    xw = jnp.dot(x_ref[...], w_ref[...], preferred_element_type=jnp.float32)
    y_ref[...] = xw * di


def _tc2_body(n, aggp_ref, y_ref, dinv_ref, b1_ref, w2_ref, zs_ref):
    a = aggp_ref[0] + aggp_ref[1] + y_ref[...]
    di = dinv_ref[...]
    h = jnp.maximum(di * a + b1_ref[...], 0.0)
    z = jnp.dot(h, w2_ref[...], preferred_element_type=jnp.float32)
    row = pl.program_id(0) * RB + lax.broadcasted_iota(jnp.int32, (RB, 1), 0)
    mask = (row < n).astype(jnp.float32)
    zs_ref[...] = di * z * mask


def kernel(x, edge_index, W1, b1, W2, b2):
    n, d_in = x.shape
    e = edge_index.shape[1]
    np_ = ((n + 2047) // 2048) * 2048
    epb = NT * EB
    nblka = (e + epb - 1) // epb
    nblka = ((nblka + 3) // 4) * 4
    epad = nblka * epb

    src = edge_index[0]
    dst = edge_index[1]
    pad = n + (jnp.arange(epad - e, dtype=src.dtype) % (np_ - n))
    src_p = jnp.concatenate([src, pad])
    dst_p = jnp.concatenate([dst, pad])
    src_a = src_p.reshape(NT, nblka, EB)
    dst_a = dst_p.reshape(NT, nblka, EB)
    ei_pk = (src_p | (dst_p << 16)).reshape(NT, nblka, EB)
    x_pad = jnp.zeros((np_, d_in), x.dtype).at[:n].set(x)

    degp = _deg_kernel(np_, nblka)(dst_a)

    np2 = np_ // RB
    y, dinv = pl.pallas_call(
        _tc1_body,
        grid=(np2,),
        in_specs=[
            pl.BlockSpec((RB, d_in), lambda i: (i, 0)),
            pl.BlockSpec((d_in, D), lambda i: (0, 0)),
            pl.BlockSpec((NC, RB, 1), lambda i: (0, i, 0)),
        ],
        out_specs=[
            pl.BlockSpec((RB, D), lambda i: (i, 0)),
            pl.BlockSpec((RB, 1), lambda i: (i, 0)),
        ],
        out_shape=[
            jax.ShapeDtypeStruct((np_, D), jnp.float32),
            jax.ShapeDtypeStruct((np_, 1), jnp.float32),
        ],
    )(x_pad, W1, degp.reshape(NC, np_, 1))

    aggp = _agg_kernel(np_, nblka)(y, ei_pk)

    zs_col = pl.pallas_call(
        functools.partial(_tc2_body, n),
        grid=(np2,),
        in_specs=[
            pl.BlockSpec((NC, RB, D), lambda i: (0, i, 0)),
            pl.BlockSpec((RB, D), lambda i: (i, 0)),
            pl.BlockSpec((RB, 1), lambda i: (i, 0)),
            pl.BlockSpec((1, D), lambda i: (0, 0)),
            pl.BlockSpec((D, 1), lambda i: (0, 0)),
        ],
        out_specs=pl.BlockSpec((RB, 1), lambda i: (i, 0)),
        out_shape=jax.ShapeDtypeStruct((np_, 1), jnp.float32),
    )(aggp, y, dinv, b1.reshape(1, D), W2)

    nblkc = 2 * nblka
    src_c = src_p.reshape(NS, nblkc, EB)
    dst_c = dst_p.reshape(NS, nblkc, EB)
    out = _sagg_kernel(np_, nblkc)(
        zs_col.reshape(np_), src_c, dst_c, dinv.reshape(np_),
        jnp.broadcast_to(b2, (LANES,)),
    )
    return out[:n]
